# baseline (speedup 1.0000x reference)
import math

import jax
import jax.numpy as jnp
from jax import lax
from jax.experimental import pallas as pl
from jax.experimental.pallas import tpu as pltpu
from jax.experimental.pallas import tpu_sc as plsc

_VOCAB = 100000
_PAD_IDX = 0
_SMOOTH = 0.1
_CONF = 1.0 - _SMOOTH
_N_TOK = 2048
_SVAL = _SMOOTH / (_VOCAB - 2)
_E_TERM = (_VOCAB - 2) * _SVAL * math.log(_SVAL) + _CONF * math.log(_CONF)

_RB = 256
_VB = 12800
_RGRID = _N_TOK // _RB
_VGRID = -(-_VOCAB // _VB)

_NW = 32
_RW = _N_TOK // _NW
_LANES = 16

_TAIL_BASE = (_VGRID - 1) * _VB
_TAIL_FULL = (_VOCAB - _TAIL_BASE) // 128
_TAIL_REM = _VOCAB - _TAIL_BASE - _TAIL_FULL * 128


def _dense_body(y_ref, tgt_ref, rs_ref, yt_ref, y0_ref, acc_ref, ytacc_ref):
    j = pl.program_id(1)
    t2d = tgt_ref[...]
    lanes = lax.broadcasted_iota(jnp.int32, (_RB, 128), 1)

    def scan_block(nslices, racc, ytacc):
        for k in range(nslices):
            s = y_ref[:, k * 128:(k + 1) * 128]
            cols = lanes + (j * _VB + k * 128)
            racc = s if racc is None else racc + s
            yta = jnp.where(t2d == cols, s, 0.0)
            ytacc = yta if ytacc is None else ytacc + yta
        return racc, ytacc

    @pl.when(j == 0)
    def _init():
        racc, ytacc = scan_block(_VB // 128, None, None)
        acc_ref[...] = racc
        ytacc_ref[...] = ytacc
        y0_ref[...] = y_ref[:, 0:1]

    @pl.when(jnp.logical_and(j > 0, j < _VGRID - 1))
    def _mid():
        racc, ytacc = scan_block(_VB // 128, acc_ref[...], ytacc_ref[...])
        acc_ref[...] = racc
        ytacc_ref[...] = ytacc

    @pl.when(j == _VGRID - 1)
    def _tail():
        racc, ytacc = scan_block(_TAIL_FULL, acc_ref[...], ytacc_ref[...])
        if _TAIL_REM:
            k = _TAIL_FULL
            s = y_ref[:, k * 128:(k + 1) * 128]
            cols = lanes + (j * _VB + k * 128)
            racc = racc + jnp.where(lanes < _TAIL_REM, s, 0.0)
            ytacc = ytacc + jnp.where(t2d == cols, s, 0.0)
        rs_ref[...] = jnp.sum(racc, axis=1, keepdims=True)
        yt_ref[...] = jnp.sum(ytacc, axis=1, keepdims=True)


def _tc_dense(y, target_2d):
    return pl.pallas_call(
        _dense_body,
        grid=(_RGRID, _VGRID),
        in_specs=[
            pl.BlockSpec((_RB, _VB), lambda i, j: (i, j)),
            pl.BlockSpec((_RB, 1), lambda i, j: (i, 0)),
        ],
        out_specs=[
            pl.BlockSpec((_RB, 1), lambda i, j: (i, 0)),
            pl.BlockSpec((_RB, 1), lambda i, j: (i, 0)),
            pl.BlockSpec((_RB, 1), lambda i, j: (i, 0)),
        ],
        out_shape=[
            jax.ShapeDtypeStruct((_N_TOK, 1), jnp.float32),
            jax.ShapeDtypeStruct((_N_TOK, 1), jnp.float32),
            jax.ShapeDtypeStruct((_N_TOK, 1), jnp.float32),
        ],
        scratch_shapes=[
            pltpu.VMEM((_RB, 128), jnp.float32),
            pltpu.VMEM((_RB, 128), jnp.float32),
        ],
    )(y, target_2d)


def _sc_body(tgt_hbm, rs_hbm, yt_hbm, y0_hbm, out_hbm,
             tgt_v, rs_v, yt_v, y0_v, acc_v):
    wid = lax.axis_index("s") * 2 + lax.axis_index("c")
    base = wid * _RW
    pltpu.sync_copy(tgt_hbm.at[pl.ds(base, _RW)], tgt_v)
    pltpu.sync_copy(rs_hbm.at[pl.ds(base, _RW)], rs_v)
    pltpu.sync_copy(yt_hbm.at[pl.ds(base, _RW)], yt_v)
    pltpu.sync_copy(y0_hbm.at[pl.ds(base, _RW)], y0_v)
    acc = jnp.zeros((_LANES,), jnp.float32)
    for k in range(_RW // _LANES):
        t16 = tgt_v[pl.ds(k * _LANES, _LANES)]
        rs = rs_v[pl.ds(k * _LANES, _LANES)]
        yt = yt_v[pl.ds(k * _LANES, _LANES)]
        y0 = y0_v[pl.ds(k * _LANES, _LANES)]
        contrib = (_E_TERM
                   - _SVAL * (rs - y0 - yt)
                   - _CONF * yt)
        acc = acc + jnp.where(t16 != _PAD_IDX, contrib, 0.0)
    acc_v[...] = acc
    pltpu.sync_copy(acc_v, out_hbm.at[pl.ds(wid * _LANES, _LANES)])


def _sc_combine(target, rs, yt, y0):
    mesh = plsc.VectorSubcoreMesh(core_axis_name="c", subcore_axis_name="s")
    fn = pl.kernel(
        _sc_body,
        out_type=jax.ShapeDtypeStruct((_NW * _LANES,), jnp.float32),
        mesh=mesh,
        compiler_params=pltpu.CompilerParams(needs_layout_passes=False),
        scratch_types=[
            pltpu.VMEM((_RW,), jnp.int32),
            pltpu.VMEM((_RW,), jnp.float32),
            pltpu.VMEM((_RW,), jnp.float32),
            pltpu.VMEM((_RW,), jnp.float32),
            pltpu.VMEM((_LANES,), jnp.float32),
        ],
    )
    return fn(target, rs, yt, y0)


def kernel(y, target):
    target = target.astype(jnp.int32)
    rs, yt, y0 = _tc_dense(y, target.reshape(_N_TOK, 1))
    sc_out = _sc_combine(target, rs.reshape(-1), yt.reshape(-1),
                         y0.reshape(-1))
    return jnp.sum(sc_out)


# speedup vs baseline: 1.0399x; 1.0399x over previous
import math

import jax
import jax.numpy as jnp
from jax import lax
from jax.experimental import pallas as pl
from jax.experimental.pallas import tpu as pltpu
from jax.experimental.pallas import tpu_sc as plsc

_VOCAB = 100000
_PAD_IDX = 0
_SMOOTH = 0.1
_CONF = 1.0 - _SMOOTH
_N_TOK = 2048
_SVAL = _SMOOTH / (_VOCAB - 2)
_E_TERM = (_VOCAB - 2) * _SVAL * math.log(_SVAL) + _CONF * math.log(_CONF)

_RB = 256
_VB = 12800
_RGRID = _N_TOK // _RB
_VGRID = -(-_VOCAB // _VB)

_NW = 32
_RW = _N_TOK // _NW
_LANES = 16
_TILE_S = 8
_TILE_L = 128

_TAIL_BASE = (_VGRID - 1) * _VB
_TAIL_FULL = (_VOCAB - _TAIL_BASE) // 128
_TAIL_REM = _VOCAB - _TAIL_BASE - _TAIL_FULL * 128


def _psum_lanes(y_ref, nslices, init=None):
    acc = init
    for k in range(nslices):
        s = y_ref[:, k * 128:(k + 1) * 128]
        acc = s if acc is None else acc + s
    return acc


def _rowsum_body(y_ref, out_ref, acc_ref):
    j = pl.program_id(1)

    @pl.when(j == 0)
    def _init():
        acc_ref[...] = _psum_lanes(y_ref, _VB // 128)

    @pl.when(jnp.logical_and(j > 0, j < _VGRID - 1))
    def _acc():
        acc_ref[...] = _psum_lanes(y_ref, _VB // 128, acc_ref[...])

    @pl.when(j == _VGRID - 1)
    def _acc_tail():
        acc = _psum_lanes(y_ref, _TAIL_FULL, acc_ref[...])
        if _TAIL_REM:
            lanes = lax.broadcasted_iota(jnp.int32, (_RB, 128), 1)
            part = y_ref[:, _TAIL_FULL * 128:(_TAIL_FULL + 1) * 128]
            acc = acc + jnp.where(lanes < _TAIL_REM, part, 0.0)
        out_ref[...] = jnp.sum(acc, axis=1, keepdims=True)


def _rowsums(y):
    return pl.pallas_call(
        _rowsum_body,
        grid=(_RGRID, _VGRID),
        in_specs=[pl.BlockSpec((_RB, _VB), lambda i, j: (i, j))],
        out_specs=pl.BlockSpec((_RB, 1), lambda i, j: (i, 0)),
        out_shape=jax.ShapeDtypeStruct((_N_TOK, 1), jnp.float32),
        scratch_shapes=[pltpu.VMEM((_RB, 128), jnp.float32)],
    )(y)


def _sc_body(y_hbm, tgt_hbm, rs_hbm, out_hbm,
             tgt_v, rs_v, buf_t, buf_0, acc_v, sem, sem2):
    wid = lax.axis_index("s") * 2 + lax.axis_index("c")
    base = wid * _RW
    pltpu.sync_copy(tgt_hbm.at[pl.ds(base, _RW)], tgt_v)
    pltpu.sync_copy(rs_hbm.at[pl.ds(base, _RW)], rs_v)
    col0 = pltpu.async_copy(
        y_hbm.at[pl.ds(base, _RW), pl.ds(0, _TILE_L)], buf_0, sem2)
    iota16 = lax.iota(jnp.int32, _LANES)
    copies = []
    for r in range(_RW):
        t16 = tgt_v[pl.ds((r // _LANES) * _LANES, _LANES)]
        t = jnp.sum(jnp.where(iota16 == (r % _LANES), t16, 0), axis=0)
        cb = pl.multiple_of((t // _TILE_L) * _TILE_L, _TILE_L)
        rg = pl.multiple_of(base + (r // _TILE_S) * _TILE_S, _TILE_S)
        copies.append(pltpu.async_copy(
            y_hbm.at[pl.ds(rg, _TILE_S), pl.ds(cb, _TILE_L)],
            buf_t.at[r], sem))
    col0.wait()
    for c in copies:
        c.wait()
    acc = jnp.zeros((_LANES,), jnp.float32)
    zeros16 = jnp.zeros((_LANES,), jnp.int32)
    for k in range(_RW // _LANES):
        t16 = tgt_v[pl.ds(k * _LANES, _LANES)]
        rows16 = iota16 + (k * _LANES)
        sub16 = lax.rem(rows16, _TILE_S)
        lanes16 = lax.rem(t16, _TILE_L)
        yt = plsc.load_gather(buf_t, [rows16, sub16, lanes16])
        y0 = plsc.load_gather(buf_0, [rows16, zeros16])
        rs = rs_v[pl.ds(k * _LANES, _LANES)]
        contrib = (_E_TERM
                   - _SVAL * (rs - y0 - yt)
                   - _CONF * yt)
        acc = acc + jnp.where(t16 != _PAD_IDX, contrib, 0.0)
    acc_v[...] = acc
    pltpu.sync_copy(acc_v, out_hbm.at[pl.ds(wid * _LANES, _LANES)])


def _sc_combine(y, target, rowsums):
    mesh = plsc.VectorSubcoreMesh(core_axis_name="c", subcore_axis_name="s")
    fn = pl.kernel(
        _sc_body,
        out_type=jax.ShapeDtypeStruct((_NW * _LANES,), jnp.float32),
        mesh=mesh,
        compiler_params=pltpu.CompilerParams(needs_layout_passes=False),
        scratch_types=[
            pltpu.VMEM((_RW,), jnp.int32),
            pltpu.VMEM((_RW,), jnp.float32),
            pltpu.VMEM((_RW, _TILE_S, _TILE_L), jnp.float32),
            pltpu.VMEM((_RW, _TILE_L), jnp.float32),
            pltpu.VMEM((_LANES,), jnp.float32),
            pltpu.SemaphoreType.DMA,
            pltpu.SemaphoreType.DMA,
        ],
    )
    return fn(y, target, rowsums)


def kernel(y, target):
    rowsums = _rowsums(y)
    sc_out = _sc_combine(y, target.astype(jnp.int32), rowsums.reshape(-1))
    return jnp.sum(sc_out)
